# baseline (speedup 1.0000x reference)
import functools

import jax
import jax.numpy as jnp
from jax import lax
from jax.experimental import pallas as pl
from jax.experimental.pallas import tpu as pltpu
from jax.experimental.pallas import tpu_sc as plsc

_B = 4096
_L = 50
_LP = 56
_V = 100000
_D = 128

_NW = 32
_BPC = 2
_CHUNK = _BPC * _LP
_BATCH_W = _B // _NW
_NCHUNK = _BATCH_W // _BPC
_NBUF = 4


@functools.partial(
    pl.kernel,
    out_type=jax.ShapeDtypeStruct((_B * _LP, _D), jnp.float32),
    mesh=plsc.VectorSubcoreMesh(core_axis_name="c", subcore_axis_name="s"),
    scratch_types=[
        pltpu.VMEM((_NCHUNK, _CHUNK), jnp.int32),
        pltpu.VMEM((_NBUF, _CHUNK, _D), jnp.float32),
        pltpu.SemaphoreType.DMA((_NBUF,)),
        pltpu.SemaphoreType.DMA((_NBUF,)),
    ],
)
def _sc_gather(tok_hbm, table_hbm, out_hbm, idx_v, rows_v, gsem, ssem):
    w = lax.axis_index("s") * 2 + lax.axis_index("c")
    pltpu.sync_copy(tok_hbm.at[w], idx_v)

    def start_gather(j, s):
        pltpu.async_copy(table_hbm.at[idx_v.at[j]], rows_v.at[s], gsem.at[s])

    for s in range(_NBUF - 1):
        start_gather(s, s)

    def quad_body(jj, _):
        for s in range(_NBUF):
            j = jj * _NBUF + s
            sn = (s + _NBUF - 1) % _NBUF

            @pl.when(j >= 1)
            def _wait_prev_store():
                pltpu.make_async_copy(
                    rows_v.at[sn], out_hbm.at[pl.ds(0, _CHUNK)],
                    ssem.at[sn]).wait()

            @pl.when(j + _NBUF - 1 < _NCHUNK)
            def _next_gather():
                start_gather(j + _NBUF - 1, sn)

            pltpu.make_async_copy(
                table_hbm.at[idx_v.at[j]], rows_v.at[s], gsem.at[s]).wait()
            b0 = w * _BATCH_W + j * _BPC
            pltpu.async_copy(
                rows_v.at[s],
                out_hbm.at[pl.ds(b0 * _LP, _CHUNK)],
                ssem.at[s])
        return _

    lax.fori_loop(0, _NCHUNK // _NBUF, quad_body, None)
    pltpu.make_async_copy(
        rows_v.at[_NBUF - 1], out_hbm.at[pl.ds(0, _CHUNK)],
        ssem.at[_NBUF - 1]).wait()


def _tc_project_emotions(x, w, bias):
    m = x.shape[0]

    def body(x_ref, w_ref, b_ref, o_ref):
        o_ref[...] = lax.dot_general(
            x_ref[...], w_ref[...], (((1,), (1,)), ((), ())),
            preferred_element_type=jnp.float32) + b_ref[...]

    return pl.pallas_call(
        body,
        grid=(1,),
        in_specs=[
            pl.BlockSpec((m, _D), lambda i: (0, 0)),
            pl.BlockSpec((_D, _D), lambda i: (0, 0)),
            pl.BlockSpec((1, _D), lambda i: (0, 0)),
        ],
        out_specs=pl.BlockSpec((m, _D), lambda i: (0, 0)),
        out_shape=jax.ShapeDtypeStruct((m, _D), jnp.float32),
    )(x, w, bias.reshape(1, _D))


_BB = 32


def _tc_project_add(g56, emo_proj, we):
    def body(x_ref, emo_ref, w_ref, o_ref):
        y = lax.dot_general(
            x_ref[...], w_ref[...], (((1,), (1,)), ((), ())),
            preferred_element_type=jnp.float32)
        for k in range(_BB):
            o_ref[k] = y[k * _LP:k * _LP + _L] + emo_ref[pl.ds(k, 1)]

    return pl.pallas_call(
        body,
        grid=(_B // _BB,),
        in_specs=[
            pl.BlockSpec((_BB * _LP, _D), lambda i: (i, 0)),
            pl.BlockSpec((_BB, _D), lambda i: (i, 0)),
            pl.BlockSpec((_D, _D), lambda i: (0, 0)),
        ],
        out_specs=pl.BlockSpec((_BB, _L, _D), lambda i: (i, 0, 0)),
        out_shape=jax.ShapeDtypeStruct((_B, _L, _D), jnp.float32),
    )(g56, emo_proj, we)


def kernel(tokens, emotions, table, W, b):
    tokens = tokens.astype(jnp.int32)
    we = W[:, :_D]
    wm = W[:, _D:]

    emo_proj = _tc_project_emotions(emotions, wm, b)
    tokp = jnp.pad(tokens, ((0, 0), (0, _LP - _L)))
    tok3 = tokp.reshape(_NW, _NCHUNK, _CHUNK)
    g56 = _sc_gather(tok3, table)
    return _tc_project_add(g56, emo_proj, we)


# speedup vs baseline: 4.5567x; 4.5567x over previous
import functools

import jax
import jax.numpy as jnp
from jax import lax
from jax.experimental import pallas as pl
from jax.experimental.pallas import tpu as pltpu
from jax.experimental.pallas import tpu_sc as plsc

_B = 4096
_L = 50
_LP = 56
_V = 100000
_D = 128

_NW = 32
_BPC = 2
_CHUNK = _BPC * _LP
_BATCH_W = _B // _NW
_NCHUNK = _BATCH_W // _BPC
_NBUF = 4


@functools.partial(
    pl.kernel,
    out_type=jax.ShapeDtypeStruct((_B * _LP, _D), jnp.float32),
    mesh=plsc.VectorSubcoreMesh(core_axis_name="c", subcore_axis_name="s"),
    scratch_types=[
        pltpu.VMEM((_NCHUNK, _CHUNK), jnp.int32),
        pltpu.VMEM((_NBUF, _CHUNK, _D), jnp.float32),
        pltpu.SemaphoreType.DMA((_NBUF,)),
        pltpu.SemaphoreType.DMA((_NBUF,)),
    ],
)
def _sc_gather(tok_hbm, table_hbm, out_hbm, idx_v, rows_v, gsem, ssem):
    w = lax.axis_index("s") * 2 + lax.axis_index("c")
    pltpu.sync_copy(tok_hbm.at[w], idx_v)

    def start_gather(j, s):
        pltpu.async_copy(table_hbm.at[idx_v.at[j]], rows_v.at[s], gsem.at[s])

    for s in range(_NBUF - 1):
        start_gather(s, s)

    def quad_body(jj, _):
        for s in range(_NBUF):
            j = jj * _NBUF + s
            sn = (s + _NBUF - 1) % _NBUF

            @pl.when(j >= 1)
            def _wait_prev_store():
                pltpu.make_async_copy(
                    rows_v.at[sn], out_hbm.at[pl.ds(0, _CHUNK)],
                    ssem.at[sn]).wait()

            @pl.when(j + _NBUF - 1 < _NCHUNK)
            def _next_gather():
                start_gather(j + _NBUF - 1, sn)

            pltpu.make_async_copy(
                table_hbm.at[idx_v.at[j]], rows_v.at[s], gsem.at[s]).wait()
            b0 = w * _BATCH_W + j * _BPC
            pltpu.async_copy(
                rows_v.at[s],
                out_hbm.at[pl.ds(b0 * _LP, _CHUNK)],
                ssem.at[s])
        return _

    lax.fori_loop(0, _NCHUNK // _NBUF, quad_body, None)
    pltpu.make_async_copy(
        rows_v.at[_NBUF - 1], out_hbm.at[pl.ds(0, _CHUNK)],
        ssem.at[_NBUF - 1]).wait()


def _tc_project_emotions(x, w, bias):
    m = x.shape[0]

    def body(x_ref, w_ref, b_ref, o_ref):
        o_ref[...] = lax.dot_general(
            x_ref[...], w_ref[...], (((1,), (1,)), ((), ())),
            preferred_element_type=jnp.float32) + b_ref[...]

    return pl.pallas_call(
        body,
        grid=(1,),
        in_specs=[
            pl.BlockSpec((m, _D), lambda i: (0, 0)),
            pl.BlockSpec((_D, _D), lambda i: (0, 0)),
            pl.BlockSpec((1, _D), lambda i: (0, 0)),
        ],
        out_specs=pl.BlockSpec((m, _D), lambda i: (0, 0)),
        out_shape=jax.ShapeDtypeStruct((m, _D), jnp.float32),
    )(x, w, bias.reshape(1, _D))


_BB = 32


def _tc_project_add(g56, emo_proj, we):
    def body(x_ref, emo_ref, w_ref, o_ref):
        y = lax.dot_general(
            x_ref[...], w_ref[...], (((1,), (1,)), ((), ())),
            preferred_element_type=jnp.float32)
        for k in range(_BB):
            o_ref[k] = y[k * _LP:k * _LP + _L] + emo_ref[pl.ds(k, 1)]

    return pl.pallas_call(
        body,
        grid=(_B // _BB,),
        in_specs=[
            pl.BlockSpec((_BB * _LP, _D), lambda i: (i, 0)),
            pl.BlockSpec((_BB, _D), lambda i: (i, 0)),
            pl.BlockSpec((_D, _D), lambda i: (0, 0)),
        ],
        out_specs=pl.BlockSpec((_BB, _L, _D), lambda i: (i, 0, 0)),
        out_shape=jax.ShapeDtypeStruct((_B, _L, _D), jnp.float32),
    )(g56, emo_proj, we)


def kernel(tokens, emotions, table, W, b):
    tokens = tokens.astype(jnp.int32)
    we = W[:, :_D]
    wm = W[:, _D:]

    emo_proj = _tc_project_emotions(emotions, wm, b)
    tokp = jnp.concatenate([tokens, tokens[:, :_LP - _L]], axis=1)
    tok3 = tokp.reshape(_NW, _NCHUNK, _CHUNK)
    g56 = _sc_gather(tok3, table)
    return _tc_project_add(g56, emo_proj, we)


# speedup vs baseline: 8.1178x; 1.7815x over previous
import functools

import jax
import jax.numpy as jnp
from jax import lax
from jax.experimental import pallas as pl
from jax.experimental.pallas import tpu as pltpu
from jax.experimental.pallas import tpu_sc as plsc

_B = 4096
_L = 50
_V = 100000
_D = 128
_R = _B * _L

_NW = 32
_CHUNK = 128
_ROWS_W = _R // _NW
_NCHUNK = _ROWS_W // _CHUNK
_NBUF = 5


@functools.partial(
    pl.kernel,
    out_type=jax.ShapeDtypeStruct((_R, _D), jnp.float32),
    mesh=plsc.VectorSubcoreMesh(core_axis_name="c", subcore_axis_name="s"),
    scratch_types=[
        pltpu.VMEM((_NCHUNK, _CHUNK), jnp.int32),
        pltpu.VMEM((_NBUF, _CHUNK, _D), jnp.float32),
        pltpu.SemaphoreType.DMA((_NBUF,)),
        pltpu.SemaphoreType.DMA((_NBUF,)),
    ],
)
def _sc_gather(tok_hbm, table_hbm, out_hbm, idx_v, rows_v, gsem, ssem):
    w = lax.axis_index("s") * 2 + lax.axis_index("c")
    pltpu.sync_copy(tok_hbm.at[w], idx_v)

    def start_gather(j, s):
        pltpu.async_copy(table_hbm.at[idx_v.at[j]], rows_v.at[s], gsem.at[s])

    for s in range(_NBUF - 1):
        start_gather(s, s)

    def ring_body(jj, _):
        for s in range(_NBUF):
            j = jj * _NBUF + s
            sn = (s + _NBUF - 1) % _NBUF

            @pl.when(j >= 1)
            def _wait_prev_store():
                pltpu.make_async_copy(
                    rows_v.at[sn], out_hbm.at[pl.ds(0, _CHUNK)],
                    ssem.at[sn]).wait()

            @pl.when(j + _NBUF - 1 < _NCHUNK)
            def _next_gather():
                start_gather(j + _NBUF - 1, sn)

            pltpu.make_async_copy(
                table_hbm.at[idx_v.at[j]], rows_v.at[s], gsem.at[s]).wait()
            pltpu.async_copy(
                rows_v.at[s],
                out_hbm.at[pl.ds(w * _ROWS_W + j * _CHUNK, _CHUNK)],
                ssem.at[s])
        return _

    lax.fori_loop(0, _NCHUNK // _NBUF, ring_body, None)
    pltpu.make_async_copy(
        rows_v.at[_NBUF - 1], out_hbm.at[pl.ds(0, _CHUNK)],
        ssem.at[_NBUF - 1]).wait()


def _tc_project_emotions(x, w, bias):
    m = x.shape[0]

    def body(x_ref, w_ref, b_ref, o_ref):
        o_ref[...] = lax.dot_general(
            x_ref[...], w_ref[...], (((1,), (1,)), ((), ())),
            preferred_element_type=jnp.float32) + b_ref[...]

    return pl.pallas_call(
        body,
        grid=(1,),
        in_specs=[
            pl.BlockSpec((m, _D), lambda i: (0, 0)),
            pl.BlockSpec((_D, _D), lambda i: (0, 0)),
            pl.BlockSpec((1, _D), lambda i: (0, 0)),
        ],
        out_specs=pl.BlockSpec((m, _D), lambda i: (0, 0)),
        out_shape=jax.ShapeDtypeStruct((m, _D), jnp.float32),
    )(x, w, bias.reshape(1, _D))


def _tc_project_add(g, emo_proj, we):
    def body(x_ref, emo_ref, w_ref, o_ref):
        o_ref[...] = lax.dot_general(
            x_ref[...], w_ref[...], (((1,), (1,)), ((), ())),
            preferred_element_type=jnp.float32) + emo_ref[...]

    return pl.pallas_call(
        body,
        grid=(_L,),
        in_specs=[
            pl.BlockSpec((_B, _D), lambda i: (i, 0)),
            pl.BlockSpec((_B, _D), lambda i: (0, 0)),
            pl.BlockSpec((_D, _D), lambda i: (0, 0)),
        ],
        out_specs=pl.BlockSpec((_B, _D), lambda i: (i, 0)),
        out_shape=jax.ShapeDtypeStruct((_R, _D), jnp.float32),
    )(g, emo_proj, we)


def kernel(tokens, emotions, table, W, b):
    tokens = tokens.astype(jnp.int32)
    we = W[:, :_D]
    wm = W[:, _D:]

    emo_proj = _tc_project_emotions(emotions, wm, b)
    tok_t = tokens.T.reshape(_NW, _NCHUNK, _CHUNK)
    g = _sc_gather(tok_t, table)
    out = _tc_project_add(g, emo_proj, we)
    return out.reshape(_L, _B, _D).transpose(1, 0, 2)
